# baseline (speedup 1.0000x reference)
import functools

import jax
import jax.numpy as jnp
from jax import lax
from jax.experimental import pallas as pl
from jax.experimental.pallas import tpu as pltpu
from jax.experimental.pallas import tpu_sc as plsc


def _make_gather(N, D, CH, NB):
    info = plsc.get_sparse_core_info()
    NC, NS = info.num_cores, info.num_subcores
    NW = NC * NS
    assert N % NW == 0
    b_per_w = N // NW
    assert b_per_w % CH == 0
    nch = b_per_w // CH
    mesh = plsc.VectorSubcoreMesh(core_axis_name="c", subcore_axis_name="s")

    @functools.partial(
        pl.kernel,
        mesh=mesh,
        out_type=jax.ShapeDtypeStruct((N, D), jnp.float32),
        scratch_types=[
            pltpu.VMEM((b_per_w,), jnp.int32),
            pltpu.VMEM((NB, CH, D), jnp.float32),
        ]
        + [pltpu.SemaphoreType.DMA] * (2 * NB),
    )
    def k(ids_hbm, table_hbm, out_hbm, idx_v, bufs, *sems):
        gsem = sems[:NB]
        wsem = sems[NB:]
        wid = lax.axis_index("s") * NC + lax.axis_index("c")
        base = wid * b_per_w
        pltpu.sync_copy(ids_hbm.at[pl.ds(base, b_per_w)], idx_v)

        gather_h = [None] * NB
        write_h = [None] * NB
        for c in range(nch):
            s = c % NB
            if write_h[s] is not None:
                write_h[s].wait()
            gather_h[s] = pltpu.async_copy(
                table_hbm.at[idx_v.at[pl.ds(c * CH, CH)]], bufs.at[s], gsem[s]
            )
            if c >= 1:
                p = (c - 1) % NB
                gather_h[p].wait()
                write_h[p] = pltpu.async_copy(
                    bufs.at[p], out_hbm.at[pl.ds(base + (c - 1) * CH, CH)], wsem[p]
                )
        p = (nch - 1) % NB
        gather_h[p].wait()
        write_h[p] = pltpu.async_copy(
            bufs.at[p], out_hbm.at[pl.ds(base + (nch - 1) * CH, CH)], wsem[p]
        )
        for h in write_h:
            h.wait()

    return k


def kernel(input_ids, table):
    B, S = input_ids.shape
    V, D = table.shape
    ids = input_ids.reshape(B * S)
    out = _make_gather(B * S, D, 32, 4)(ids, table)
    return out.reshape(B, S, D)


# speedup vs baseline: 1.0070x; 1.0070x over previous
import functools

import jax
import jax.numpy as jnp
from jax import lax
from jax.experimental import pallas as pl
from jax.experimental.pallas import tpu as pltpu
from jax.experimental.pallas import tpu_sc as plsc


def _make_gather(B, S, D, CH, NB):
    info = plsc.get_sparse_core_info()
    NC, NS = info.num_cores, info.num_subcores
    NW = NC * NS
    N = B * S
    assert N % NW == 0
    b_per_w = N // NW
    assert b_per_w % CH == 0 and S % b_per_w == 0
    w_per_b = S // b_per_w
    nch = b_per_w // CH
    mesh = plsc.VectorSubcoreMesh(core_axis_name="c", subcore_axis_name="s")

    @functools.partial(
        pl.kernel,
        mesh=mesh,
        out_type=jax.ShapeDtypeStruct((B, S, D), jnp.float32),
        scratch_types=[
            pltpu.VMEM((b_per_w,), jnp.int32),
            pltpu.VMEM((NB, CH, D), jnp.float32),
        ]
        + [pltpu.SemaphoreType.DMA] * (2 * NB),
    )
    def k(ids_hbm, table_hbm, out_hbm, idx_v, bufs, *sems):
        gsem = sems[:NB]
        wsem = sems[NB:]
        wid = lax.axis_index("s") * NC + lax.axis_index("c")
        b = wid // w_per_b
        off = (wid % w_per_b) * b_per_w
        pltpu.sync_copy(ids_hbm.at[b, pl.ds(off, b_per_w)], idx_v)

        gather_h = [None] * NB
        write_h = [None] * NB
        for c in range(nch):
            s = c % NB
            if write_h[s] is not None:
                write_h[s].wait()
            gather_h[s] = pltpu.async_copy(
                table_hbm.at[idx_v.at[pl.ds(c * CH, CH)]], bufs.at[s], gsem[s]
            )
            if c >= 1:
                p = (c - 1) % NB
                gather_h[p].wait()
                write_h[p] = pltpu.async_copy(
                    bufs.at[p], out_hbm.at[b, pl.ds(off + (c - 1) * CH, CH)], wsem[p]
                )
        p = (nch - 1) % NB
        gather_h[p].wait()
        write_h[p] = pltpu.async_copy(
            bufs.at[p], out_hbm.at[b, pl.ds(off + (nch - 1) * CH, CH)], wsem[p]
        )
        for h in write_h:
            h.wait()

    return k


def kernel(input_ids, table):
    B, S = input_ids.shape
    V, D = table.shape
    return _make_gather(B, S, D, 64, 2)(input_ids, table)


# speedup vs baseline: 1.0121x; 1.0051x over previous
import functools

import jax
import jax.numpy as jnp
from jax import lax
from jax.experimental import pallas as pl
from jax.experimental.pallas import tpu as pltpu
from jax.experimental.pallas import tpu_sc as plsc


def _make_gather(B, S, D, CH, NB):
    info = plsc.get_sparse_core_info()
    NC, NS = info.num_cores, info.num_subcores
    NW = NC * NS
    N = B * S
    assert N % NW == 0
    b_per_w = N // NW
    assert b_per_w % CH == 0 and S % b_per_w == 0
    w_per_b = S // b_per_w
    nch = b_per_w // CH
    mesh = plsc.VectorSubcoreMesh(core_axis_name="c", subcore_axis_name="s")

    @functools.partial(
        pl.kernel,
        mesh=mesh,
        out_type=jax.ShapeDtypeStruct((B, S, D), jnp.float32),
        scratch_types=[
            pltpu.VMEM((b_per_w,), jnp.int32),
            pltpu.VMEM((NB, CH, D), jnp.float32),
        ]
        + [pltpu.SemaphoreType.DMA] * (2 * NB),
    )
    def k(ids_hbm, table_hbm, out_hbm, idx_v, bufs, *sems):
        gsem = sems[:NB]
        wsem = sems[NB:]
        wid = lax.axis_index("s") * NC + lax.axis_index("c")
        b = wid // w_per_b
        off = (wid % w_per_b) * b_per_w
        half = b_per_w // 2
        pltpu.sync_copy(ids_hbm.at[b, pl.ds(off, half)], idx_v.at[pl.ds(0, half)])

        gather_h = [None] * NB
        write_h = [None] * NB
        rest_h = None
        for c in range(nch):
            s = c % NB
            if write_h[s] is not None:
                write_h[s].wait()
            gather_h[s] = pltpu.async_copy(
                table_hbm.at[idx_v.at[pl.ds(c * CH, CH)]], bufs.at[s], gsem[s]
            )
            if c == 0:
                rest_h = pltpu.async_copy(
                    ids_hbm.at[b, pl.ds(off + half, half)],
                    idx_v.at[pl.ds(half, half)],
                    wsem[NB - 1],
                )
                rest_h.wait()
            if c >= 1:
                p = (c - 1) % NB
                gather_h[p].wait()
                write_h[p] = pltpu.async_copy(
                    bufs.at[p], out_hbm.at[b, pl.ds(off + (c - 1) * CH, CH)], wsem[p]
                )
        p = (nch - 1) % NB
        gather_h[p].wait()
        write_h[p] = pltpu.async_copy(
            bufs.at[p], out_hbm.at[b, pl.ds(off + (nch - 1) * CH, CH)], wsem[p]
        )
        for h in write_h:
            h.wait()

    return k


def kernel(input_ids, table):
    B, S = input_ids.shape
    V, D = table.shape
    return _make_gather(B, S, D, 64, 2)(input_ids, table)
